# Initial kernel scaffold; baseline (speedup 1.0000x reference)
import jax
import jax.numpy as jnp
from jax.experimental import pallas as pl


def kernel(x, edge_index, alpha):
    raise NotImplementedError("write your pallas kernel here")


# speedup vs baseline: 32.0169x; 32.0169x over previous
import functools

import jax
import jax.numpy as jnp
from jax import lax
from jax.experimental import pallas as pl
from jax.experimental.pallas import tpu as pltpu
from jax.experimental.pallas import tpu_sc as plsc

N_NODES = 10000
D = 128
HD = D // 2
E = 320000

NC = 2
NS = 16
C = 128
NCH = 80
NCH2 = 160
E_PAD = NC * NS * NCH * C
N_PAD = 10240
RPT = N_PAD // NS

_sc_mesh = plsc.VectorSubcoreMesh(core_axis_name="c", subcore_axis_name="s")


@functools.partial(
    pl.kernel,
    out_type=jax.ShapeDtypeStruct((NC, N_PAD), jnp.float32),
    mesh=_sc_mesh,
    scratch_types=[
        pltpu.VMEM((NCH, C), jnp.int32),
        pltpu.VMEM((C,), jnp.float32),
        pltpu.VMEM((RPT,), jnp.float32),
        pltpu.VMEM_SHARED((N_PAD,), jnp.float32),
    ],
)
def _deg_kernel(col_hbm, degp_hbm, col_v, ones_v, stage_v, deg_sh):
    c = lax.axis_index("c")
    s = lax.axis_index("s")
    one16 = jnp.ones((16,), jnp.float32)
    zero16 = jnp.zeros((16,), jnp.float32)
    for i in range(C // 16):
        ones_v[pl.ds(i * 16, 16)] = one16
    for i in range(RPT // 16):
        stage_v[pl.ds(i * 16, 16)] = zero16
    pltpu.sync_copy(stage_v, deg_sh.at[pl.ds(s * RPT, RPT)])
    pltpu.sync_copy(col_hbm.at[c, s], col_v)
    plsc.subcore_barrier()

    def body(j, _):
        pltpu.sync_copy(ones_v, deg_sh.at[col_v.at[j]], add=True)
        return ()

    lax.fori_loop(0, NCH, body, ())
    plsc.subcore_barrier()
    pltpu.sync_copy(deg_sh.at[pl.ds(s * RPT, RPT)], stage_v)
    pltpu.sync_copy(stage_v, degp_hbm.at[c, pl.ds(s * RPT, RPT)])


@functools.partial(
    pl.kernel,
    out_type=jax.ShapeDtypeStruct((NC, N_PAD, HD), jnp.float32),
    mesh=_sc_mesh,
    scratch_types=[
        pltpu.VMEM((NCH2, C), jnp.int32),
        pltpu.VMEM((NCH2, C), jnp.int32),
        pltpu.VMEM((2, C, HD), jnp.float32),
        pltpu.VMEM((C, HD), jnp.float32),
        pltpu.VMEM_SHARED((N_PAD, HD), jnp.float32),
        pltpu.SemaphoreType.DMA,
        pltpu.SemaphoreType.DMA,
    ],
    compiler_params=pltpu.CompilerParams(use_tc_tiling_on_sc=False),
)
def _agg_kernel(xs_hbm, row_hbm, col_hbm, outp_hbm,
                row_v, col_v, msg_v, stage_v, acc_sh, sem0, sem1):
    c = lax.axis_index("c")
    s = lax.axis_index("s")
    zero16 = jnp.zeros((16,), jnp.float32)

    def zbody(i, _):
        for jj in range(HD // 16):
            stage_v[i, pl.ds(jj * 16, 16)] = zero16
        return ()

    lax.fori_loop(0, C, zbody, ())
    for piece in range(RPT // C):
        pltpu.sync_copy(stage_v, acc_sh.at[pl.ds(s * RPT + piece * C, C)])
    pltpu.sync_copy(row_hbm.at[s], row_v)
    pltpu.sync_copy(col_hbm.at[s], col_v)
    plsc.subcore_barrier()

    xsc = xs_hbm.at[c]

    pltpu.async_copy(xsc.at[row_v.at[0]], msg_v.at[0], sem0)

    def body(i, _):
        j = 2 * i
        pltpu.async_copy(xsc.at[row_v.at[j + 1]], msg_v.at[1], sem1)
        pltpu.make_async_copy(xsc.at[row_v.at[j]], msg_v.at[0], sem0).wait()
        pltpu.sync_copy(msg_v.at[0], acc_sh.at[col_v.at[j]], add=True)

        @pl.when(j + 2 < NCH2)
        def _start_next():
            pltpu.async_copy(xsc.at[row_v.at[j + 2]], msg_v.at[0], sem0)

        pltpu.make_async_copy(xsc.at[row_v.at[j + 1]], msg_v.at[1], sem1).wait()
        pltpu.sync_copy(msg_v.at[1], acc_sh.at[col_v.at[j + 1]], add=True)
        return ()

    lax.fori_loop(0, NCH2 // 2, body, ())
    plsc.subcore_barrier()
    for piece in range(RPT // C):
        r0 = s * RPT + piece * C
        pltpu.sync_copy(acc_sh.at[pl.ds(r0, C)], stage_v)
        pltpu.sync_copy(stage_v, outp_hbm.at[c, pl.ds(r0, C)])


def _prescale_body(degp_ref, x_ref, xs_ref):
    deg = degp_ref[:, 0:1] + degp_ref[:, 1:2]
    dis = jnp.where(deg > 0, lax.rsqrt(deg), 0.0)
    xs = x_ref[...] * dis
    xs_ref[0] = xs[:, :HD]
    xs_ref[1] = xs[:, HD:]


_prescale = pl.pallas_call(
    _prescale_body,
    grid=(10,),
    in_specs=[
        pl.BlockSpec((1000, 2), lambda i: (i, 0)),
        pl.BlockSpec((1000, D), lambda i: (i, 0)),
    ],
    out_specs=pl.BlockSpec((2, 1000, HD), lambda i: (0, i, 0)),
    out_shape=jax.ShapeDtypeStruct((NC, N_NODES, HD), jnp.float32),
)


def _combine_body(alpha_ref, degp_ref, p_ref, out_ref):
    deg = degp_ref[:, 0:1] + degp_ref[:, 1:2]
    dis = jnp.where(deg > 0, lax.rsqrt(deg), 0.0)
    acc = jnp.concatenate([p_ref[0], p_ref[1]], axis=1)
    out_ref[...] = (alpha_ref[0, 0] * dis) * acc


_combine = pl.pallas_call(
    _combine_body,
    grid=(10,),
    in_specs=[
        pl.BlockSpec(memory_space=pltpu.SMEM),
        pl.BlockSpec((1000, 2), lambda i: (i, 0)),
        pl.BlockSpec((2, 1000, HD), lambda i: (0, i, 0)),
    ],
    out_specs=pl.BlockSpec((1000, D), lambda i: (i, 0)),
    out_shape=jax.ShapeDtypeStruct((N_NODES, D), jnp.float32),
)


def kernel(x, edge_index, alpha):
    row = edge_index[0].astype(jnp.int32)
    col = edge_index[1].astype(jnp.int32)
    pad_n = E_PAD - E
    pad_row = jnp.arange(pad_n, dtype=jnp.int32) % N_NODES
    pad_col = N_NODES + jnp.arange(pad_n, dtype=jnp.int32) % (N_PAD - N_NODES)
    row_flat = jnp.concatenate([row, pad_row])
    col_flat = jnp.concatenate([col, pad_col])

    degp = _deg_kernel(col_flat.reshape(NC, NS, NCH, C))
    degp_t = degp[:, :N_NODES].T
    xs = _prescale(degp_t, x)
    outp = _agg_kernel(xs,
                       row_flat.reshape(NS, NCH2, C),
                       col_flat.reshape(NS, NCH2, C))
    return _combine(alpha.reshape(1, 1), degp_t, outp)
